# baseline (speedup 1.0000x reference)
import functools

import jax
import jax.numpy as jnp
from jax import lax
from jax.experimental import pallas as pl
from jax.experimental.pallas import tpu as pltpu
from jax.experimental.pallas import tpu_sc as plsc

N_NODES = 50000
N_EDGES = 800000
HID = 16
FE = 8
LANES = FE * HID

NC = 2
NS = 16
NW = NC * NS

EPAD = 819200
E2 = 2 * EPAD
EROWS = EPAD // FE
BR = 512
NBE = EROWS // BR
NROWS = N_NODES // FE
BNR = 256
NBN = -(-NROWS // BNR)
MO_OFF = 51200
ACC_ROWS = 2 * MO_OFF
ZROWS = ACC_ROWS // NS
PROWS = ACC_ROWS // FE
MO_BLK = MO_OFF // FE // BNR


EA = 425984
EB = EPAD - EA
EAF = EA // FE
EBF = EB // FE

_G_CH = 16
_S_CH = 8


@functools.cache
def _sc_mesh():
    return plsc.VectorSubcoreMesh(
        core_axis_name="c", subcore_axis_name="s",
        num_cores=NC, num_subcores=NS)


_SC_PARAMS = pltpu.CompilerParams(use_tc_tiling_on_sc=False)


_G_ROWS = _G_CH * 128


def _make_gather_body(cpt, outer_n):
    def body(h_hbm, idx_hbm, out_hbm, idx_v, rows_v, sg0, sg1, so0, so1):
        wid = lax.axis_index("s") * NC + lax.axis_index("c")
        chunk0 = wid * cpt
        sg = (sg0, sg1)
        so = (so0, so1)

        def fire(g, b):
            cb = chunk0 + g * _G_CH
            pltpu.sync_copy(idx_hbm.at[pl.ds(cb, _G_CH)], idx_v.at[b])
            for j in range(_G_CH):
                pltpu.async_copy(
                    h_hbm.at[idx_v.at[b, j]],
                    rows_v.at[b, pl.ds(j * 128, 128)],
                    sg[b])

        def wait_and_flush(g, b):
            pltpu.make_async_copy(
                h_hbm.at[pl.ds(0, _G_ROWS)], rows_v.at[b], sg[b]).wait()
            cb = chunk0 + g * _G_CH
            pltpu.async_copy(rows_v.at[b],
                             out_hbm.at[pl.ds(cb * 128, _G_ROWS)], so[b])

        def drain_out(g, b):
            cb = chunk0 + g * _G_CH
            pltpu.make_async_copy(
                rows_v.at[b], out_hbm.at[pl.ds(cb * 128, _G_ROWS)], so[b]).wait()

        fire(0, 0)

        def outer(go, carry):
            for b in (1, 0):
                g = 2 * go + (1 if b == 1 else 2)

                @pl.when(g < outer_n)
                def _():
                    @pl.when(go > 0)
                    def _():
                        drain_out(g - 2, b)
                    fire(g, b)

                @pl.when(g - 1 < outer_n)
                def _():
                    wait_and_flush(g - 1, 1 - b)
            return carry

        lax.fori_loop(0, (outer_n + 2) // 2, outer, 0)
        drain_out(outer_n - 2, (outer_n - 2) % 2)
        drain_out(outer_n - 1, (outer_n - 1) % 2)

    return body


@functools.cache
def _sc_gather_kernel(rows):
    cpt = rows // 128 // NW
    assert cpt % _G_CH == 0
    return pl.kernel(
        _make_gather_body(cpt, cpt // _G_CH),
        out_type=jax.ShapeDtypeStruct((rows, HID), jnp.float32),
        mesh=_sc_mesh(),
        scratch_types=[
            pltpu.VMEM((2, _G_CH, 128), jnp.int32),
            pltpu.VMEM((2, _G_ROWS, HID), jnp.float32),
            pltpu.SemaphoreType.DMA,
            pltpu.SemaphoreType.DMA,
            pltpu.SemaphoreType.DMA,
            pltpu.SemaphoreType.DMA,
        ],
        compiler_params=_SC_PARAMS,
    )


def _sc_gather(h_rows, idx, rows):
    return _sc_gather_kernel(rows)(h_rows, idx)


@functools.cache
def _sc_scatter_kernel():
    return pl.kernel(
        _sc_scatter_body,
        out_type=jax.ShapeDtypeStruct((NC, ACC_ROWS, HID), jnp.float32),
        mesh=_sc_mesh(),
        scratch_types=[
            pltpu.VMEM((_S_CH, 128), jnp.int32),
            pltpu.VMEM((_S_CH * 128, HID), jnp.float32),
            pltpu.VMEM_SHARED((ACC_ROWS, HID), jnp.float32),
            pltpu.SemaphoreType.DMA,
        ],
        compiler_params=_SC_PARAMS,
    )


def _sc_scatter(wsa, wea, wsb, web, ia_mi, ia_mo, ib_mi, ib_mo, zeros):
    return _sc_scatter_kernel()(wsa, wea, wsb, web, ia_mi, ia_mo, ib_mi, ib_mo,
                                zeros)


def _sc_scatter_body(wsa_hbm, wea_hbm, wsb_hbm, web_hbm,
                     ia_mi_hbm, ia_mo_hbm, ib_mi_hbm, ib_mo_hbm,
                     zeros_hbm, out_hbm, idx_v, vals_v, acc_sh, sem):
    c = lax.axis_index("c")
    s = lax.axis_index("s")
    wid = s * NC + c

    pltpu.sync_copy(zeros_hbm, acc_sh.at[pl.ds(s * ZROWS, ZROWS)])
    plsc.subcore_barrier()

    def run_pass(vals_hbm, idx_hbm, cpt):
        def outer(g, carry):
            cb = wid * cpt + g * _S_CH
            pltpu.sync_copy(idx_hbm.at[pl.ds(cb, _S_CH)], idx_v)
            pltpu.sync_copy(vals_hbm.at[pl.ds(cb * 128, _S_CH * 128)], vals_v)
            descs = []
            for j in range(_S_CH):
                descs.append(pltpu.async_copy(
                    vals_v.at[pl.ds(j * 128, 128)],
                    acc_sh.at[idx_v.at[j]],
                    sem, add=True))
            for d in descs:
                d.wait()
            return carry
        lax.fori_loop(0, cpt // _S_CH, outer, 0)

    cpt_a = EA // 128 // NW
    cpt_b = EB // 128 // NW
    run_pass(wsa_hbm, ia_mi_hbm, cpt_a)
    run_pass(wea_hbm, ia_mo_hbm, cpt_a)
    run_pass(wsb_hbm, ib_mi_hbm, cpt_b)
    run_pass(web_hbm, ib_mo_hbm, cpt_b)

    plsc.subcore_barrier()
    pltpu.sync_copy(acc_sh.at[pl.ds(s * ZROWS, ZROWS)],
                    out_hbm.at[c, pl.ds(s * ZROWS, ZROWS)])


_INV_H = 1.0 / HID
_BF = jnp.bfloat16


def _bdot(x, w):
    return jnp.dot(x.astype(_BF), w, preferred_element_type=jnp.float32)


def _lnf(y, bdsum, g, b):
    mean = _bdot(y, bdsum) * _INV_H
    c = y - mean
    v = _bdot(c * c, bdsum) * _INV_H
    return g * c * jax.lax.rsqrt(v + 1e-5) + b


def _layer(h, w, bdsum, aux_ref, i):
    b = aux_ref[3 * i:3 * i + 1, :]
    g = aux_ref[3 * i + 1:3 * i + 2, :]
    bb = aux_ref[3 * i + 2:3 * i + 3, :]
    return jax.nn.relu(_lnf(_bdot(h, w) + b, bdsum, g, bb))


def _in_embed_body(xf_ref, w_ref, aux_ref, out_ref):
    w = w_ref[0:LANES, :]
    bdsum = w_ref[LANES:2 * LANES, :]
    h = _bdot(xf_ref[...], w)
    h = h + aux_ref[0:1, :]
    out_ref[...] = jax.nn.relu(_lnf(h, bdsum, aux_ref[1:2, :], aux_ref[2:3, :]))


def _edge_hidden(gs, ge, w_ref, aux_ref):
    bdsum = w_ref[640:768, :]
    h = _bdot(gs, w_ref[0:128, :]) + _bdot(ge, w_ref[128:256, :])
    b = aux_ref[0:1, :]
    g = aux_ref[1:2, :]
    bb = aux_ref[2:3, :]
    h = jax.nn.relu(_lnf(h + b, bdsum, g, bb))
    h = _layer(h, w_ref[256:384, :], bdsum, aux_ref, 1)
    h = _layer(h, w_ref[384:512, :], bdsum, aux_ref, 2)
    return h


def _edge_logit_b(h, w_ref, aux_ref):
    return _bdot(h, w_ref[512:640, :]) + aux_ref[9:10, :]


def _make_edge_body(off):
    def body(gs_ref, ge_ref, w_ref, aux_ref, ws_ref, we_ref):
        i = pl.program_id(0)
        gs = gs_ref[...]
        ge = ge_ref[...]
        h = _edge_hidden(gs, ge, w_ref, aux_ref)
        e = jax.nn.sigmoid(_edge_logit_b(h, w_ref, aux_ref))
        r = lax.broadcasted_iota(jnp.int32, (BR, LANES), 0)
        l = lax.broadcasted_iota(jnp.int32, (BR, LANES), 1)
        eid = off + (i * BR + r) * FE + (l >> 4)
        mask = eid < N_EDGES
        ws_ref[...] = jnp.where(mask, e * gs, 0.0)
        we_ref[...] = jnp.where(mask, e * ge, 0.0)
    return body


def _make_final_edge_body(off):
    def body(gs_ref, ge_ref, we_ref, auxe_ref, wp_ref, auxp_ref,
             sel_ref, a_ref, ep_ref):
        i = pl.program_id(0)
        gs = gs_ref[...]
        ge = ge_ref[...]
        sel = sel_ref[...]
        ha = _edge_hidden(gs, ge, we_ref, auxe_ref)
        alog8 = jnp.dot(_edge_logit_b(ha, we_ref, auxe_ref), sel,
                        preferred_element_type=jnp.float32)
        a_ref[...] = jax.nn.sigmoid(alog8)
        hp = _edge_hidden(gs, ge, wp_ref, auxp_ref)
        ep8 = jnp.dot(_edge_logit_b(hp, wp_ref, auxp_ref), sel,
                      preferred_element_type=jnp.float32)
        r = lax.broadcasted_iota(jnp.int32, (BR, FE), 0)
        g = lax.broadcasted_iota(jnp.int32, (BR, FE), 1)
        eid = off + (i * BR + r) * FE + g
        ep_ref[...] = jnp.where(eid < N_EDGES, ep8, -jnp.inf)
    return body


def _node_mlp_body(mi0_ref, mi1_ref, mo0_ref, mo1_ref, h_ref, w_ref,
                   aux_ref, out_ref):
    mi = mi0_ref[0] + mi1_ref[0]
    mo = mo0_ref[0] + mo1_ref[0]
    h0 = h_ref[...]
    bdsum = w_ref[768:896, :]
    h = (_bdot(mi, w_ref[0:128, :]) + _bdot(mo, w_ref[128:256, :])
         + _bdot(h0, w_ref[256:384, :]))
    h = jax.nn.relu(_lnf(h + aux_ref[0:1, :], bdsum,
                         aux_ref[1:2, :], aux_ref[2:3, :]))
    h = _layer(h, w_ref[384:512, :], bdsum, aux_ref, 1)
    h = _layer(h, w_ref[512:640, :], bdsum, aux_ref, 2)
    h = _layer(h, w_ref[640:768, :], bdsum, aux_ref, 3)
    out_ref[...] = h + h0


def _gsum_body(h_ref, out_ref):
    i = pl.program_id(0)

    @pl.when(i == 0)
    def _():
        out_ref[...] = jnp.zeros_like(out_ref)

    r = lax.broadcasted_iota(jnp.int32, (BNR, LANES), 0)
    blk = jnp.where(i * BNR + r < NROWS, h_ref[...], 0.0)
    out_ref[...] += jnp.sum(blk, axis=0, keepdims=True)


def _glob_body(gsf_ref, t_ref, wcat_ref, aux_ref, w3_ref, b3_ref,
               graw_ref, gsig_ref):
    h = jnp.dot(gsf_ref[...], t_ref[...], preferred_element_type=jnp.float32)
    for i, (r0, r1) in enumerate(((0, 16), (16, 32), (32, 48))):
        w = wcat_ref[r0:r1, :]
        b = aux_ref[3 * i:3 * i + 1, :]
        g = aux_ref[3 * i + 1:3 * i + 2, :]
        bb = aux_ref[3 * i + 2:3 * i + 3, :]
        hlin = jnp.dot(h, w, preferred_element_type=jnp.float32) + b
        mu = jnp.mean(hlin, axis=-1, keepdims=True)
        va = jnp.mean((hlin - mu) * (hlin - mu), axis=-1, keepdims=True)
        h = jax.nn.relu(g * (hlin - mu) * jax.lax.rsqrt(va + 1e-5) + bb)
    graw = jnp.dot(h, w3_ref[...], preferred_element_type=jnp.float32) + b3_ref[...]
    graw_ref[...] = graw
    gsig_ref[...] = jax.nn.sigmoid(graw)


def _lsm_pass1_body(ep_ref, g00_ref, m_ref, s_ref, acc_ref):
    i = pl.program_id(0)

    @pl.when(i == 0)
    def _():
        acc_ref[0] = g00_ref[0, 0]
        acc_ref[1] = 1.0

    m_old = acc_ref[0]
    s_old = acc_ref[1]
    blk = ep_ref[...]
    bm = jnp.max(blk)
    m_new = jnp.maximum(m_old, bm)
    s_new = s_old * jnp.exp(m_old - m_new) + jnp.sum(jnp.exp(blk - m_new))
    acc_ref[0] = m_new
    acc_ref[1] = s_new

    @pl.when(i == pl.num_programs(0) - 1)
    def _():
        m_ref[0, 0] = m_new
        s_ref[0, 0] = s_new


def _lsm_pass2_body(ep_ref, g00_ref, m_ref, s_ref, lsm_ref, lsm0_ref):
    z = m_ref[0, 0] + jnp.log(s_ref[0, 0])
    lsm_ref[...] = ep_ref[...] - z

    @pl.when(pl.program_id(0) == 0)
    def _():
        lsm0_ref[0, 0] = g00_ref[0, 0] - z


def _bd(w):
    return jnp.kron(jnp.eye(FE, dtype=jnp.float32), w)


def _bdsum():
    return jnp.kron(jnp.eye(FE, dtype=jnp.float32),
                    jnp.ones((HID, HID), jnp.float32))


def _tile_rows(p, names):
    return jnp.stack([jnp.tile(v, FE) for v in names], axis=0)


def _pack_edge(p):
    w0 = p["l0"]["W"]
    head = jnp.outer(p["l3"]["W"][:, 0], jnp.ones((HID,), jnp.float32))
    wbig = jnp.concatenate([
        _bd(w0[:HID]), _bd(w0[HID:]), _bd(p["l1"]["W"]), _bd(p["l2"]["W"]),
        _bd(head), _bdsum()], axis=0)
    rows = []
    for i in range(3):
        rows += [p["l%d" % i]["b"], p["ln%d" % i]["g"], p["ln%d" % i]["b"]]
    rows.append(jnp.full((HID,), p["l3"]["b"][0], jnp.float32))
    aux = _tile_rows(p, rows)
    return wbig, aux


def _pack_node(p):
    w0 = p["l0"]["W"]
    wbig = jnp.concatenate([
        _bd(w0[0:HID]), _bd(w0[HID:2 * HID]), _bd(w0[2 * HID:3 * HID]),
        _bd(p["l1"]["W"]), _bd(p["l2"]["W"]), _bd(p["l3"]["W"]),
        _bdsum()], axis=0)
    rows = []
    for i in range(4):
        rows += [p["l%d" % i]["b"], p["ln%d" % i]["g"], p["ln%d" % i]["b"]]
    return wbig, _tile_rows(p, rows)


def _pack_glob(p):
    wcat = jnp.concatenate([p["l0"]["W"], p["l1"]["W"], p["l2"]["W"]], axis=0)
    rows = []
    for i in range(3):
        rows += [p["l%d" % i]["b"], p["ln%d" % i]["g"], p["ln%d" % i]["b"]]
    return wcat, jnp.stack(rows, axis=0), p["l3"]["W"], p["l3"]["b"][None, :]


_TC_PARAMS = pltpu.CompilerParams(dimension_semantics=("arbitrary",))


def _wspec(shape):
    return pl.BlockSpec(shape, lambda i: (0,) * len(shape))


def kernel(params, x, edge_index):
    start = edge_index[0]
    end = edge_index[1]
    zpad = jnp.zeros((EPAD - N_EDGES,), jnp.int32)
    startp = jnp.concatenate([start, zpad])
    endp = jnp.concatenate([end, zpad])

    idx_ga = jnp.concatenate([startp[:EA], endp[:EA]]).reshape(-1, 128)
    idx_gb = jnp.concatenate([startp[EA:], endp[EA:]]).reshape(-1, 128)
    ia_mi = endp[:EA].reshape(-1, 128)
    ia_mo = (startp[:EA] + MO_OFF).reshape(-1, 128)
    ib_mi = endp[EA:].reshape(-1, 128)
    ib_mo = (startp[EA:] + MO_OFF).reshape(-1, 128)
    zrows = jnp.zeros((ZROWS, HID), jnp.float32)

    wbig_e, aux_e = _pack_edge(params["edge"])
    wbig_p, aux_p = _pack_edge(params["prune"])
    wbig_n, aux_n = _pack_node(params["node"])
    wbig_e = wbig_e.astype(_BF)
    wbig_p = wbig_p.astype(_BF)
    wbig_n = wbig_n.astype(_BF)
    wcat_g, aux_g, w3_g, b3_g = _pack_glob(params["glob"])
    sel = jnp.kron(jnp.eye(FE, dtype=jnp.float32),
                   jnp.zeros((HID, 1), jnp.float32).at[0, 0].set(1.0))
    tmat = jnp.tile(jnp.eye(HID, dtype=jnp.float32), (FE, 1))

    xf = jnp.pad(x, ((0, 0), (0, HID - 3))).reshape(NROWS, LANES)
    w16 = jnp.pad(params["in_lin"]["W"], ((0, HID - 3), (0, 0)))
    wbig_i = jnp.concatenate([_bd(w16), _bdsum()], axis=0).astype(_BF)
    aux_i = _tile_rows(None, [params["in_lin"]["b"], params["in_ln"]["g"],
                              params["in_ln"]["b"]])

    hf = pl.pallas_call(
        _in_embed_body,
        grid=(NBN,),
        in_specs=[pl.BlockSpec((BNR, LANES), lambda i: (i, 0)),
                  _wspec((2 * LANES, LANES)), _wspec((3, LANES))],
        out_specs=pl.BlockSpec((BNR, LANES), lambda i: (i, 0)),
        out_shape=jax.ShapeDtypeStruct((NROWS, LANES), jnp.float32),
        compiler_params=_TC_PARAMS,
    )(xf, wbig_i, aux_i)

    def edge_call(nblk, off):
        return pl.pallas_call(
            _make_edge_body(off),
            grid=(nblk,),
            in_specs=[pl.BlockSpec((BR, LANES), lambda i: (i, 0)),
                      pl.BlockSpec((BR, LANES), lambda i: (nblk + i, 0)),
                      _wspec((768, LANES)), _wspec((10, LANES))],
            out_specs=[pl.BlockSpec((BR, LANES), lambda i: (i, 0)),
                       pl.BlockSpec((BR, LANES), lambda i: (i, 0))],
            out_shape=[jax.ShapeDtypeStruct((nblk * BR, LANES), jnp.float32),
                       jax.ShapeDtypeStruct((nblk * BR, LANES), jnp.float32)],
            compiler_params=_TC_PARAMS,
        )

    edge_a = edge_call(EAF // BR, 0)
    edge_b = edge_call(EBF // BR, EA)

    node_mlp = pl.pallas_call(
        _node_mlp_body,
        grid=(NBN,),
        in_specs=[pl.BlockSpec((1, BNR, LANES), lambda i: (0, i, 0)),
                  pl.BlockSpec((1, BNR, LANES), lambda i: (1, i, 0)),
                  pl.BlockSpec((1, BNR, LANES), lambda i: (0, MO_BLK + i, 0)),
                  pl.BlockSpec((1, BNR, LANES), lambda i: (1, MO_BLK + i, 0)),
                  pl.BlockSpec((BNR, LANES), lambda i: (i, 0)),
                  _wspec((896, LANES)), _wspec((12, LANES))],
        out_specs=pl.BlockSpec((BNR, LANES), lambda i: (i, 0)),
        out_shape=jax.ShapeDtypeStruct((NROWS, LANES), jnp.float32),
        compiler_params=_TC_PARAMS,
    )

    def final_call(nblk, off):
        return pl.pallas_call(
            _make_final_edge_body(off),
            grid=(nblk,),
            in_specs=[pl.BlockSpec((BR, LANES), lambda i: (i, 0)),
                      pl.BlockSpec((BR, LANES), lambda i: (nblk + i, 0)),
                      _wspec((768, LANES)), _wspec((10, LANES)),
                      _wspec((768, LANES)), _wspec((10, LANES)),
                      _wspec((LANES, FE))],
            out_specs=[pl.BlockSpec((BR, FE), lambda i: (i, 0)),
                       pl.BlockSpec((BR, FE), lambda i: (i, 0))],
            out_shape=[jax.ShapeDtypeStruct((nblk * BR, FE), jnp.float32),
                       jax.ShapeDtypeStruct((nblk * BR, FE), jnp.float32)],
            compiler_params=_TC_PARAMS,
        )

    for _ in range(3):
        h_rows = hf.reshape(N_NODES, HID)
        ga = _sc_gather(h_rows, idx_ga, 2 * EA).reshape(2 * EAF, LANES)
        gb = _sc_gather(h_rows, idx_gb, 2 * EB).reshape(2 * EBF, LANES)
        wsa, wea = edge_a(ga, ga, wbig_e, aux_e)
        wsb, web = edge_b(gb, gb, wbig_e, aux_e)
        part = _sc_scatter(wsa.reshape(EA, HID), wea.reshape(EA, HID),
                           wsb.reshape(EB, HID), web.reshape(EB, HID),
                           ia_mi, ia_mo, ib_mi, ib_mo, zrows)
        pf = part.reshape(NC, PROWS, LANES)
        hf = node_mlp(pf, pf, pf, pf, hf, wbig_n, aux_n)

    h_rows = hf.reshape(N_NODES, HID)
    ga = _sc_gather(h_rows, idx_ga, 2 * EA).reshape(2 * EAF, LANES)
    gb = _sc_gather(h_rows, idx_gb, 2 * EB).reshape(2 * EBF, LANES)
    a_sig_a, ep_a = final_call(EAF // BR, 0)(
        ga, ga, wbig_e, aux_e, wbig_p, aux_p, sel)
    a_sig_b, ep_b = final_call(EBF // BR, EA)(
        gb, gb, wbig_e, aux_e, wbig_p, aux_p, sel)
    a_sig = jnp.concatenate([a_sig_a, a_sig_b], axis=0)
    ep = jnp.concatenate([ep_a, ep_b], axis=0)

    gsf = pl.pallas_call(
        _gsum_body,
        grid=(NBN,),
        in_specs=[pl.BlockSpec((BNR, LANES), lambda i: (i, 0))],
        out_specs=pl.BlockSpec((1, LANES), lambda i: (0, 0)),
        out_shape=jax.ShapeDtypeStruct((1, LANES), jnp.float32),
        compiler_params=_TC_PARAMS,
    )(hf)

    graw, gsig = pl.pallas_call(
        _glob_body,
        out_shape=[jax.ShapeDtypeStruct((1, 3), jnp.float32),
                   jax.ShapeDtypeStruct((1, 3), jnp.float32)],
    )(gsf, tmat, wcat_g, aux_g, w3_g, b3_g)

    g00 = graw[:, 0:1]
    _sspec = pl.BlockSpec(memory_space=pltpu.SMEM)
    BSR = 4096
    m, s = pl.pallas_call(
        _lsm_pass1_body,
        grid=(EROWS // BSR,),
        in_specs=[pl.BlockSpec((BSR, FE), lambda i: (i, 0)), _sspec],
        out_specs=[_sspec, _sspec],
        out_shape=[jax.ShapeDtypeStruct((1, 1), jnp.float32),
                   jax.ShapeDtypeStruct((1, 1), jnp.float32)],
        scratch_shapes=[pltpu.SMEM((2,), jnp.float32)],
        compiler_params=_TC_PARAMS,
    )(ep, g00)

    lsm, lsm0 = pl.pallas_call(
        _lsm_pass2_body,
        grid=(EROWS // BSR,),
        in_specs=[pl.BlockSpec((BSR, FE), lambda i: (i, 0)), _sspec,
                  _sspec, _sspec],
        out_specs=[pl.BlockSpec((BSR, FE), lambda i: (i, 0)), _sspec],
        out_shape=[jax.ShapeDtypeStruct((EROWS, FE), jnp.float32),
                   jax.ShapeDtypeStruct((1, 1), jnp.float32)],
        compiler_params=_TC_PARAMS,
    )(ep, g00, m, s)

    out_lsm = jnp.concatenate([lsm0[0], lsm.reshape(-1)[:N_EDGES]])
    return (out_lsm, gsig[0, 1:2], a_sig.reshape(-1)[:N_EDGES], gsig[0, 2:3])


# speedup vs baseline: 1.0046x; 1.0046x over previous
import functools

import jax
import jax.numpy as jnp
from jax import lax
from jax.experimental import pallas as pl
from jax.experimental.pallas import tpu as pltpu
from jax.experimental.pallas import tpu_sc as plsc

N_NODES = 50000
N_EDGES = 800000
HID = 16
FE = 8
LANES = FE * HID

NC = 2
NS = 16
NW = NC * NS

EPAD = 819200
E2 = 2 * EPAD
EROWS = EPAD // FE
BR = 512
NBE = EROWS // BR
NROWS = N_NODES // FE
BNR = 256
NBN = -(-NROWS // BNR)
MO_OFF = 51200
ACC_ROWS = 2 * MO_OFF
ZROWS = ACC_ROWS // NS
PROWS = ACC_ROWS // FE
MO_BLK = MO_OFF // FE // BNR


EA = 425984
EB = EPAD - EA
EAF = EA // FE
EBF = EB // FE

_G_CH = 16
_S_CH = 8


@functools.cache
def _sc_mesh():
    return plsc.VectorSubcoreMesh(
        core_axis_name="c", subcore_axis_name="s",
        num_cores=NC, num_subcores=NS)


_SC_PARAMS = pltpu.CompilerParams(use_tc_tiling_on_sc=False)


_G_ROWS = _G_CH * 128


def _make_gather_body(cpt, outer_n):
    def body(h_hbm, idx_hbm, out_hbm, idx_v, rows_v, sg0, sg1, so0, so1):
        wid = lax.axis_index("s") * NC + lax.axis_index("c")
        chunk0 = wid * cpt
        sg = (sg0, sg1)
        so = (so0, so1)

        def fire(g, b):
            cb = chunk0 + g * _G_CH
            pltpu.sync_copy(idx_hbm.at[pl.ds(cb, _G_CH)], idx_v.at[b])
            for j in range(_G_CH):
                pltpu.async_copy(
                    h_hbm.at[idx_v.at[b, j]],
                    rows_v.at[b, pl.ds(j * 128, 128)],
                    sg[b])

        def wait_and_flush(g, b):
            pltpu.make_async_copy(
                h_hbm.at[pl.ds(0, _G_ROWS)], rows_v.at[b], sg[b]).wait()
            cb = chunk0 + g * _G_CH
            pltpu.async_copy(rows_v.at[b],
                             out_hbm.at[pl.ds(cb * 128, _G_ROWS)], so[b])

        def drain_out(g, b):
            cb = chunk0 + g * _G_CH
            pltpu.make_async_copy(
                rows_v.at[b], out_hbm.at[pl.ds(cb * 128, _G_ROWS)], so[b]).wait()

        fire(0, 0)

        def outer(go, carry):
            for b in (1, 0):
                g = 2 * go + (1 if b == 1 else 2)

                @pl.when(g < outer_n)
                def _():
                    @pl.when(go > 0)
                    def _():
                        drain_out(g - 2, b)
                    fire(g, b)

                @pl.when(g - 1 < outer_n)
                def _():
                    wait_and_flush(g - 1, 1 - b)
            return carry

        lax.fori_loop(0, (outer_n + 2) // 2, outer, 0)
        drain_out(outer_n - 2, (outer_n - 2) % 2)
        drain_out(outer_n - 1, (outer_n - 1) % 2)

    return body


@functools.cache
def _sc_gather_kernel(rows):
    cpt = rows // 128 // NW
    assert cpt % _G_CH == 0
    return pl.kernel(
        _make_gather_body(cpt, cpt // _G_CH),
        out_type=jax.ShapeDtypeStruct((rows, HID), jnp.float32),
        mesh=_sc_mesh(),
        scratch_types=[
            pltpu.VMEM((2, _G_CH, 128), jnp.int32),
            pltpu.VMEM((2, _G_ROWS, HID), jnp.float32),
            pltpu.SemaphoreType.DMA,
            pltpu.SemaphoreType.DMA,
            pltpu.SemaphoreType.DMA,
            pltpu.SemaphoreType.DMA,
        ],
        compiler_params=_SC_PARAMS,
    )


def _sc_gather(h_rows, idx, rows):
    return _sc_gather_kernel(rows)(h_rows, idx)


@functools.cache
def _sc_scatter_kernel():
    return pl.kernel(
        _sc_scatter_body,
        out_type=jax.ShapeDtypeStruct((NC, ACC_ROWS, HID), jnp.float32),
        mesh=_sc_mesh(),
        scratch_types=[
            pltpu.VMEM((_S_CH, 128), jnp.int32),
            pltpu.VMEM((_S_CH * 128, HID), jnp.float32),
            pltpu.VMEM_SHARED((ACC_ROWS, HID), jnp.float32),
            pltpu.SemaphoreType.DMA,
        ],
        compiler_params=_SC_PARAMS,
    )


def _sc_scatter(wsa, wea, wsb, web, ia_mi, ia_mo, ib_mi, ib_mo, zeros):
    return _sc_scatter_kernel()(wsa, wea, wsb, web, ia_mi, ia_mo, ib_mi, ib_mo,
                                zeros)


def _sc_scatter_body(wsa_hbm, wea_hbm, wsb_hbm, web_hbm,
                     ia_mi_hbm, ia_mo_hbm, ib_mi_hbm, ib_mo_hbm,
                     zeros_hbm, out_hbm, idx_v, vals_v, acc_sh, sem):
    c = lax.axis_index("c")
    s = lax.axis_index("s")
    wid = s * NC + c

    pltpu.sync_copy(zeros_hbm, acc_sh.at[pl.ds(s * ZROWS, ZROWS)])
    plsc.subcore_barrier()

    def run_pass(vals_hbm, idx_hbm, cpt):
        def outer(g, carry):
            cb = wid * cpt + g * _S_CH
            pltpu.sync_copy(idx_hbm.at[pl.ds(cb, _S_CH)], idx_v)
            pltpu.sync_copy(vals_hbm.at[pl.ds(cb * 128, _S_CH * 128)], vals_v)
            descs = []
            for j in range(_S_CH):
                descs.append(pltpu.async_copy(
                    vals_v.at[pl.ds(j * 128, 128)],
                    acc_sh.at[idx_v.at[j]],
                    sem, add=True))
            for d in descs:
                d.wait()
            return carry
        lax.fori_loop(0, cpt // _S_CH, outer, 0)

    cpt_a = EA // 128 // NW
    cpt_b = EB // 128 // NW
    run_pass(wsa_hbm, ia_mi_hbm, cpt_a)
    run_pass(wea_hbm, ia_mo_hbm, cpt_a)
    run_pass(wsb_hbm, ib_mi_hbm, cpt_b)
    run_pass(web_hbm, ib_mo_hbm, cpt_b)

    plsc.subcore_barrier()
    pltpu.sync_copy(acc_sh.at[pl.ds(s * ZROWS, ZROWS)],
                    out_hbm.at[c, pl.ds(s * ZROWS, ZROWS)])


_INV_H = 1.0 / HID
_BF = jnp.bfloat16


def _bdot(x, w):
    return jnp.dot(x, w, preferred_element_type=jnp.float32)


def _lnf(y, bdsum, g, b):
    mean = _bdot(y, bdsum) * _INV_H
    c = y - mean
    v = _bdot(c * c, bdsum) * _INV_H
    return g * c * jax.lax.rsqrt(v + 1e-5) + b


def _layer(h, w, bdsum, aux_ref, i):
    b = aux_ref[3 * i:3 * i + 1, :]
    g = aux_ref[3 * i + 1:3 * i + 2, :]
    bb = aux_ref[3 * i + 2:3 * i + 3, :]
    return jax.nn.relu(_lnf(_bdot(h, w) + b, bdsum, g, bb))


def _in_embed_body(xf_ref, w_ref, aux_ref, out_ref):
    w = w_ref[0:LANES, :]
    bdsum = w_ref[LANES:2 * LANES, :]
    h = _bdot(xf_ref[...], w)
    h = h + aux_ref[0:1, :]
    out_ref[...] = jax.nn.relu(_lnf(h, bdsum, aux_ref[1:2, :], aux_ref[2:3, :]))


def _edge_hidden(gs, ge, w_ref, aux_ref):
    bdsum = w_ref[640:768, :]
    h = _bdot(gs, w_ref[0:128, :]) + _bdot(ge, w_ref[128:256, :])
    b = aux_ref[0:1, :]
    g = aux_ref[1:2, :]
    bb = aux_ref[2:3, :]
    h = jax.nn.relu(_lnf(h + b, bdsum, g, bb))
    h = _layer(h, w_ref[256:384, :], bdsum, aux_ref, 1)
    h = _layer(h, w_ref[384:512, :], bdsum, aux_ref, 2)
    return h


def _edge_logit_b(h, w_ref, aux_ref):
    return _bdot(h, w_ref[512:640, :]) + aux_ref[9:10, :]


def _make_edge_body(off):
    def body(gs_ref, ge_ref, w_ref, aux_ref, ws_ref, we_ref):
        i = pl.program_id(0)
        gs = gs_ref[...]
        ge = ge_ref[...]
        h = _edge_hidden(gs, ge, w_ref, aux_ref)
        e = jax.nn.sigmoid(_edge_logit_b(h, w_ref, aux_ref))
        r = lax.broadcasted_iota(jnp.int32, (BR, LANES), 0)
        l = lax.broadcasted_iota(jnp.int32, (BR, LANES), 1)
        eid = off + (i * BR + r) * FE + (l >> 4)
        mask = eid < N_EDGES
        ws_ref[...] = jnp.where(mask, e * gs, 0.0)
        we_ref[...] = jnp.where(mask, e * ge, 0.0)
    return body


def _make_final_edge_body(off):
    def body(gs_ref, ge_ref, we_ref, auxe_ref, wp_ref, auxp_ref,
             sel_ref, a_ref, ep_ref):
        i = pl.program_id(0)
        gs = gs_ref[...]
        ge = ge_ref[...]
        sel = sel_ref[...]
        ha = _edge_hidden(gs, ge, we_ref, auxe_ref)
        alog8 = jnp.dot(_edge_logit_b(ha, we_ref, auxe_ref), sel,
                        preferred_element_type=jnp.float32)
        a_ref[...] = jax.nn.sigmoid(alog8)
        hp = _edge_hidden(gs, ge, wp_ref, auxp_ref)
        ep8 = jnp.dot(_edge_logit_b(hp, wp_ref, auxp_ref), sel,
                      preferred_element_type=jnp.float32)
        r = lax.broadcasted_iota(jnp.int32, (BR, FE), 0)
        g = lax.broadcasted_iota(jnp.int32, (BR, FE), 1)
        eid = off + (i * BR + r) * FE + g
        ep_ref[...] = jnp.where(eid < N_EDGES, ep8, -jnp.inf)
    return body


def _node_mlp_body(mi0_ref, mi1_ref, mo0_ref, mo1_ref, h_ref, w_ref,
                   aux_ref, out_ref):
    mi = mi0_ref[0] + mi1_ref[0]
    mo = mo0_ref[0] + mo1_ref[0]
    h0 = h_ref[...]
    bdsum = w_ref[768:896, :]
    h = (_bdot(mi, w_ref[0:128, :]) + _bdot(mo, w_ref[128:256, :])
         + _bdot(h0, w_ref[256:384, :]))
    h = jax.nn.relu(_lnf(h + aux_ref[0:1, :], bdsum,
                         aux_ref[1:2, :], aux_ref[2:3, :]))
    h = _layer(h, w_ref[384:512, :], bdsum, aux_ref, 1)
    h = _layer(h, w_ref[512:640, :], bdsum, aux_ref, 2)
    h = _layer(h, w_ref[640:768, :], bdsum, aux_ref, 3)
    out_ref[...] = h + h0


def _gsum_body(h_ref, out_ref):
    i = pl.program_id(0)

    @pl.when(i == 0)
    def _():
        out_ref[...] = jnp.zeros_like(out_ref)

    r = lax.broadcasted_iota(jnp.int32, (BNR, LANES), 0)
    blk = jnp.where(i * BNR + r < NROWS, h_ref[...], 0.0)
    out_ref[...] += jnp.sum(blk, axis=0, keepdims=True)


def _glob_body(gsf_ref, t_ref, wcat_ref, aux_ref, w3_ref, b3_ref,
               graw_ref, gsig_ref):
    h = jnp.dot(gsf_ref[...], t_ref[...], preferred_element_type=jnp.float32)
    for i, (r0, r1) in enumerate(((0, 16), (16, 32), (32, 48))):
        w = wcat_ref[r0:r1, :]
        b = aux_ref[3 * i:3 * i + 1, :]
        g = aux_ref[3 * i + 1:3 * i + 2, :]
        bb = aux_ref[3 * i + 2:3 * i + 3, :]
        hlin = jnp.dot(h, w, preferred_element_type=jnp.float32) + b
        mu = jnp.mean(hlin, axis=-1, keepdims=True)
        va = jnp.mean((hlin - mu) * (hlin - mu), axis=-1, keepdims=True)
        h = jax.nn.relu(g * (hlin - mu) * jax.lax.rsqrt(va + 1e-5) + bb)
    graw = jnp.dot(h, w3_ref[...], preferred_element_type=jnp.float32) + b3_ref[...]
    graw_ref[...] = graw
    gsig_ref[...] = jax.nn.sigmoid(graw)


def _lsm_pass1_body(ep_ref, g00_ref, m_ref, s_ref, acc_ref):
    i = pl.program_id(0)

    @pl.when(i == 0)
    def _():
        acc_ref[0] = g00_ref[0, 0]
        acc_ref[1] = 1.0

    m_old = acc_ref[0]
    s_old = acc_ref[1]
    blk = ep_ref[...]
    bm = jnp.max(blk)
    m_new = jnp.maximum(m_old, bm)
    s_new = s_old * jnp.exp(m_old - m_new) + jnp.sum(jnp.exp(blk - m_new))
    acc_ref[0] = m_new
    acc_ref[1] = s_new

    @pl.when(i == pl.num_programs(0) - 1)
    def _():
        m_ref[0, 0] = m_new
        s_ref[0, 0] = s_new


def _lsm_pass2_body(ep_ref, g00_ref, m_ref, s_ref, lsm_ref, lsm0_ref):
    z = m_ref[0, 0] + jnp.log(s_ref[0, 0])
    lsm_ref[...] = ep_ref[...] - z

    @pl.when(pl.program_id(0) == 0)
    def _():
        lsm0_ref[0, 0] = g00_ref[0, 0] - z


def _bd(w):
    return jnp.kron(jnp.eye(FE, dtype=jnp.float32), w)


def _bdsum():
    return jnp.kron(jnp.eye(FE, dtype=jnp.float32),
                    jnp.ones((HID, HID), jnp.float32))


def _tile_rows(p, names):
    return jnp.stack([jnp.tile(v, FE) for v in names], axis=0)


def _pack_edge(p):
    w0 = p["l0"]["W"]
    head = jnp.outer(p["l3"]["W"][:, 0], jnp.ones((HID,), jnp.float32))
    wbig = jnp.concatenate([
        _bd(w0[:HID]), _bd(w0[HID:]), _bd(p["l1"]["W"]), _bd(p["l2"]["W"]),
        _bd(head), _bdsum()], axis=0)
    rows = []
    for i in range(3):
        rows += [p["l%d" % i]["b"], p["ln%d" % i]["g"], p["ln%d" % i]["b"]]
    rows.append(jnp.full((HID,), p["l3"]["b"][0], jnp.float32))
    aux = _tile_rows(p, rows)
    return wbig, aux


def _pack_node(p):
    w0 = p["l0"]["W"]
    wbig = jnp.concatenate([
        _bd(w0[0:HID]), _bd(w0[HID:2 * HID]), _bd(w0[2 * HID:3 * HID]),
        _bd(p["l1"]["W"]), _bd(p["l2"]["W"]), _bd(p["l3"]["W"]),
        _bdsum()], axis=0)
    rows = []
    for i in range(4):
        rows += [p["l%d" % i]["b"], p["ln%d" % i]["g"], p["ln%d" % i]["b"]]
    return wbig, _tile_rows(p, rows)


def _pack_glob(p):
    wcat = jnp.concatenate([p["l0"]["W"], p["l1"]["W"], p["l2"]["W"]], axis=0)
    rows = []
    for i in range(3):
        rows += [p["l%d" % i]["b"], p["ln%d" % i]["g"], p["ln%d" % i]["b"]]
    return wcat, jnp.stack(rows, axis=0), p["l3"]["W"], p["l3"]["b"][None, :]


_TC_PARAMS = pltpu.CompilerParams(dimension_semantics=("arbitrary",))


def _wspec(shape):
    return pl.BlockSpec(shape, lambda i: (0,) * len(shape))


def kernel(params, x, edge_index):
    start = edge_index[0]
    end = edge_index[1]
    zpad = jnp.zeros((EPAD - N_EDGES,), jnp.int32)
    startp = jnp.concatenate([start, zpad])
    endp = jnp.concatenate([end, zpad])

    idx_ga = jnp.concatenate([startp[:EA], endp[:EA]]).reshape(-1, 128)
    idx_gb = jnp.concatenate([startp[EA:], endp[EA:]]).reshape(-1, 128)
    ia_mi = endp[:EA].reshape(-1, 128)
    ia_mo = (startp[:EA] + MO_OFF).reshape(-1, 128)
    ib_mi = endp[EA:].reshape(-1, 128)
    ib_mo = (startp[EA:] + MO_OFF).reshape(-1, 128)
    zrows = jnp.zeros((ZROWS, HID), jnp.float32)

    wbig_e, aux_e = _pack_edge(params["edge"])
    wbig_p, aux_p = _pack_edge(params["prune"])
    wbig_n, aux_n = _pack_node(params["node"])
    wcat_g, aux_g, w3_g, b3_g = _pack_glob(params["glob"])
    sel = jnp.kron(jnp.eye(FE, dtype=jnp.float32),
                   jnp.zeros((HID, 1), jnp.float32).at[0, 0].set(1.0))
    tmat = jnp.tile(jnp.eye(HID, dtype=jnp.float32), (FE, 1))

    xf = jnp.pad(x, ((0, 0), (0, HID - 3))).reshape(NROWS, LANES)
    w16 = jnp.pad(params["in_lin"]["W"], ((0, HID - 3), (0, 0)))
    wbig_i = jnp.concatenate([_bd(w16), _bdsum()], axis=0)
    aux_i = _tile_rows(None, [params["in_lin"]["b"], params["in_ln"]["g"],
                              params["in_ln"]["b"]])

    hf = pl.pallas_call(
        _in_embed_body,
        grid=(NBN,),
        in_specs=[pl.BlockSpec((BNR, LANES), lambda i: (i, 0)),
                  _wspec((2 * LANES, LANES)), _wspec((3, LANES))],
        out_specs=pl.BlockSpec((BNR, LANES), lambda i: (i, 0)),
        out_shape=jax.ShapeDtypeStruct((NROWS, LANES), jnp.float32),
        compiler_params=_TC_PARAMS,
    )(xf, wbig_i, aux_i)

    def edge_call(nblk, off):
        return pl.pallas_call(
            _make_edge_body(off),
            grid=(nblk,),
            in_specs=[pl.BlockSpec((BR, LANES), lambda i: (i, 0)),
                      pl.BlockSpec((BR, LANES), lambda i: (nblk + i, 0)),
                      _wspec((768, LANES)), _wspec((10, LANES))],
            out_specs=[pl.BlockSpec((BR, LANES), lambda i: (i, 0)),
                       pl.BlockSpec((BR, LANES), lambda i: (i, 0))],
            out_shape=[jax.ShapeDtypeStruct((nblk * BR, LANES), jnp.float32),
                       jax.ShapeDtypeStruct((nblk * BR, LANES), jnp.float32)],
            compiler_params=_TC_PARAMS,
        )

    edge_a = edge_call(EAF // BR, 0)
    edge_b = edge_call(EBF // BR, EA)

    node_mlp = pl.pallas_call(
        _node_mlp_body,
        grid=(NBN,),
        in_specs=[pl.BlockSpec((1, BNR, LANES), lambda i: (0, i, 0)),
                  pl.BlockSpec((1, BNR, LANES), lambda i: (1, i, 0)),
                  pl.BlockSpec((1, BNR, LANES), lambda i: (0, MO_BLK + i, 0)),
                  pl.BlockSpec((1, BNR, LANES), lambda i: (1, MO_BLK + i, 0)),
                  pl.BlockSpec((BNR, LANES), lambda i: (i, 0)),
                  _wspec((896, LANES)), _wspec((12, LANES))],
        out_specs=pl.BlockSpec((BNR, LANES), lambda i: (i, 0)),
        out_shape=jax.ShapeDtypeStruct((NROWS, LANES), jnp.float32),
        compiler_params=_TC_PARAMS,
    )

    def final_call(nblk, off):
        return pl.pallas_call(
            _make_final_edge_body(off),
            grid=(nblk,),
            in_specs=[pl.BlockSpec((BR, LANES), lambda i: (i, 0)),
                      pl.BlockSpec((BR, LANES), lambda i: (nblk + i, 0)),
                      _wspec((768, LANES)), _wspec((10, LANES)),
                      _wspec((768, LANES)), _wspec((10, LANES)),
                      _wspec((LANES, FE))],
            out_specs=[pl.BlockSpec((BR, FE), lambda i: (i, 0)),
                       pl.BlockSpec((BR, FE), lambda i: (i, 0))],
            out_shape=[jax.ShapeDtypeStruct((nblk * BR, FE), jnp.float32),
                       jax.ShapeDtypeStruct((nblk * BR, FE), jnp.float32)],
            compiler_params=_TC_PARAMS,
        )

    for _ in range(3):
        h_rows = hf.reshape(N_NODES, HID)
        ga = _sc_gather(h_rows, idx_ga, 2 * EA).reshape(2 * EAF, LANES)
        gb = _sc_gather(h_rows, idx_gb, 2 * EB).reshape(2 * EBF, LANES)
        wsa, wea = edge_a(ga, ga, wbig_e, aux_e)
        wsb, web = edge_b(gb, gb, wbig_e, aux_e)
        part = _sc_scatter(wsa.reshape(EA, HID), wea.reshape(EA, HID),
                           wsb.reshape(EB, HID), web.reshape(EB, HID),
                           ia_mi, ia_mo, ib_mi, ib_mo, zrows)
        pf = part.reshape(NC, PROWS, LANES)
        hf = node_mlp(pf, pf, pf, pf, hf, wbig_n, aux_n)

    h_rows = hf.reshape(N_NODES, HID)
    ga = _sc_gather(h_rows, idx_ga, 2 * EA).reshape(2 * EAF, LANES)
    gb = _sc_gather(h_rows, idx_gb, 2 * EB).reshape(2 * EBF, LANES)
    a_sig_a, ep_a = final_call(EAF // BR, 0)(
        ga, ga, wbig_e, aux_e, wbig_p, aux_p, sel)
    a_sig_b, ep_b = final_call(EBF // BR, EA)(
        gb, gb, wbig_e, aux_e, wbig_p, aux_p, sel)
    a_sig = jnp.concatenate([a_sig_a, a_sig_b], axis=0)
    ep = jnp.concatenate([ep_a, ep_b], axis=0)

    gsf = pl.pallas_call(
        _gsum_body,
        grid=(NBN,),
        in_specs=[pl.BlockSpec((BNR, LANES), lambda i: (i, 0))],
        out_specs=pl.BlockSpec((1, LANES), lambda i: (0, 0)),
        out_shape=jax.ShapeDtypeStruct((1, LANES), jnp.float32),
        compiler_params=_TC_PARAMS,
    )(hf)

    graw, gsig = pl.pallas_call(
        _glob_body,
        out_shape=[jax.ShapeDtypeStruct((1, 3), jnp.float32),
                   jax.ShapeDtypeStruct((1, 3), jnp.float32)],
    )(gsf, tmat, wcat_g, aux_g, w3_g, b3_g)

    g00 = graw[:, 0:1]
    _sspec = pl.BlockSpec(memory_space=pltpu.SMEM)
    BSR = 4096
    m, s = pl.pallas_call(
        _lsm_pass1_body,
        grid=(EROWS // BSR,),
        in_specs=[pl.BlockSpec((BSR, FE), lambda i: (i, 0)), _sspec],
        out_specs=[_sspec, _sspec],
        out_shape=[jax.ShapeDtypeStruct((1, 1), jnp.float32),
                   jax.ShapeDtypeStruct((1, 1), jnp.float32)],
        scratch_shapes=[pltpu.SMEM((2,), jnp.float32)],
        compiler_params=_TC_PARAMS,
    )(ep, g00)

    lsm, lsm0 = pl.pallas_call(
        _lsm_pass2_body,
        grid=(EROWS // BSR,),
        in_specs=[pl.BlockSpec((BSR, FE), lambda i: (i, 0)), _sspec,
                  _sspec, _sspec],
        out_specs=[pl.BlockSpec((BSR, FE), lambda i: (i, 0)), _sspec],
        out_shape=[jax.ShapeDtypeStruct((EROWS, FE), jnp.float32),
                   jax.ShapeDtypeStruct((1, 1), jnp.float32)],
        compiler_params=_TC_PARAMS,
    )(ep, g00, m, s)

    out_lsm = jnp.concatenate([lsm0[0], lsm.reshape(-1)[:N_EDGES]])
    return (out_lsm, gsig[0, 1:2], a_sig.reshape(-1)[:N_EDGES], gsig[0, 2:3])


# speedup vs baseline: 1.0084x; 1.0038x over previous
import functools

import jax
import jax.numpy as jnp
from jax import lax
from jax.experimental import pallas as pl
from jax.experimental.pallas import tpu as pltpu
from jax.experimental.pallas import tpu_sc as plsc

N_NODES = 50000
N_EDGES = 800000
HID = 16
FE = 8
LANES = FE * HID

NC = 2
NS = 16
NW = NC * NS

EPAD = 819200
E2 = 2 * EPAD
EROWS = EPAD // FE
BR = 512
NBE = EROWS // BR
NROWS = N_NODES // FE
BNR = 256
NBN = -(-NROWS // BNR)
MO_OFF = 51200
ACC_ROWS = 2 * MO_OFF
ZROWS = ACC_ROWS // NS
PROWS = ACC_ROWS // FE
MO_BLK = MO_OFF // FE // BNR


EA = 425984
EB = EPAD - EA
EAF = EA // FE
EBF = EB // FE

_G_CH = 16
_S_CH = 8


@functools.cache
def _sc_mesh():
    return plsc.VectorSubcoreMesh(
        core_axis_name="c", subcore_axis_name="s",
        num_cores=NC, num_subcores=NS)


_SC_PARAMS = pltpu.CompilerParams(use_tc_tiling_on_sc=False)


_G_ROWS = _G_CH * 128


def _make_gather_body(cpt, outer_n):
    def body(h_hbm, idx_hbm, out_hbm, idx_v, rows_v, sg0, sg1, so0, so1):
        wid = lax.axis_index("s") * NC + lax.axis_index("c")
        chunk0 = wid * cpt
        sg = (sg0, sg1)
        so = (so0, so1)

        def fire(g, b):
            cb = chunk0 + g * _G_CH
            pltpu.sync_copy(idx_hbm.at[pl.ds(cb, _G_CH)], idx_v.at[b])
            for j in range(_G_CH):
                pltpu.async_copy(
                    h_hbm.at[idx_v.at[b, j]],
                    rows_v.at[b, pl.ds(j * 128, 128)],
                    sg[b])

        def wait_and_flush(g, b):
            pltpu.make_async_copy(
                h_hbm.at[pl.ds(0, _G_ROWS)], rows_v.at[b], sg[b]).wait()
            cb = chunk0 + g * _G_CH
            pltpu.async_copy(rows_v.at[b],
                             out_hbm.at[pl.ds(cb * 128, _G_ROWS)], so[b])

        def drain_out(g, b):
            cb = chunk0 + g * _G_CH
            pltpu.make_async_copy(
                rows_v.at[b], out_hbm.at[pl.ds(cb * 128, _G_ROWS)], so[b]).wait()

        fire(0, 0)

        def outer(go, carry):
            for b in (1, 0):
                g = 2 * go + (1 if b == 1 else 2)

                @pl.when(g < outer_n)
                def _():
                    @pl.when(go > 0)
                    def _():
                        drain_out(g - 2, b)
                    fire(g, b)

                @pl.when(g - 1 < outer_n)
                def _():
                    wait_and_flush(g - 1, 1 - b)
            return carry

        lax.fori_loop(0, (outer_n + 2) // 2, outer, 0)
        drain_out(outer_n - 2, (outer_n - 2) % 2)
        drain_out(outer_n - 1, (outer_n - 1) % 2)

    return body


@functools.cache
def _sc_gather_kernel(rows):
    cpt = rows // 128 // NW
    assert cpt % _G_CH == 0
    return pl.kernel(
        _make_gather_body(cpt, cpt // _G_CH),
        out_type=jax.ShapeDtypeStruct((rows, HID), jnp.float32),
        mesh=_sc_mesh(),
        scratch_types=[
            pltpu.VMEM((2, _G_CH, 128), jnp.int32),
            pltpu.VMEM((2, _G_ROWS, HID), jnp.float32),
            pltpu.SemaphoreType.DMA,
            pltpu.SemaphoreType.DMA,
            pltpu.SemaphoreType.DMA,
            pltpu.SemaphoreType.DMA,
        ],
        compiler_params=_SC_PARAMS,
    )


def _sc_gather(h_rows, idx, rows):
    return _sc_gather_kernel(rows)(h_rows, idx)


@functools.cache
def _sc_scatter_kernel():
    return pl.kernel(
        _sc_scatter_body,
        out_type=jax.ShapeDtypeStruct((NC, ACC_ROWS, HID), jnp.float32),
        mesh=_sc_mesh(),
        scratch_types=[
            pltpu.VMEM((_S_CH, 128), jnp.int32),
            pltpu.VMEM((_S_CH * 128, HID), jnp.float32),
            pltpu.VMEM_SHARED((ACC_ROWS, HID), jnp.float32),
            pltpu.SemaphoreType.DMA,
        ],
        compiler_params=_SC_PARAMS,
    )


def _sc_scatter(wsa, wea, wsb, web, ia_mi, ia_mo, ib_mi, ib_mo, zeros):
    return _sc_scatter_kernel()(wsa, wea, wsb, web, ia_mi, ia_mo, ib_mi, ib_mo,
                                zeros)


def _sc_scatter_body(wsa_hbm, wea_hbm, wsb_hbm, web_hbm,
                     ia_mi_hbm, ia_mo_hbm, ib_mi_hbm, ib_mo_hbm,
                     zeros_hbm, out_hbm, idx_v, vals_v, acc_sh, sem):
    c = lax.axis_index("c")
    s = lax.axis_index("s")
    wid = s * NC + c

    pltpu.sync_copy(zeros_hbm, acc_sh.at[pl.ds(s * ZROWS, ZROWS)])
    plsc.subcore_barrier()

    def run_pass(vals_hbm, idx_hbm, cpt):
        def outer(g, carry):
            cb = wid * cpt + g * _S_CH
            pltpu.sync_copy(idx_hbm.at[pl.ds(cb, _S_CH)], idx_v)
            pltpu.sync_copy(vals_hbm.at[pl.ds(cb * 128, _S_CH * 128)], vals_v)
            descs = []
            for j in range(_S_CH):
                descs.append(pltpu.async_copy(
                    vals_v.at[pl.ds(j * 128, 128)],
                    acc_sh.at[idx_v.at[j]],
                    sem, add=True))
            for d in descs:
                d.wait()
            return carry
        lax.fori_loop(0, cpt // _S_CH, outer, 0)

    cpt_a = EA // 128 // NW
    cpt_b = EB // 128 // NW
    run_pass(wsa_hbm, ia_mi_hbm, cpt_a)
    run_pass(wea_hbm, ia_mo_hbm, cpt_a)
    run_pass(wsb_hbm, ib_mi_hbm, cpt_b)
    run_pass(web_hbm, ib_mo_hbm, cpt_b)

    plsc.subcore_barrier()
    pltpu.sync_copy(acc_sh.at[pl.ds(s * ZROWS, ZROWS)],
                    out_hbm.at[c, pl.ds(s * ZROWS, ZROWS)])


_INV_H = 1.0 / HID


def _bdot(x, w):
    return jnp.dot(x, w, preferred_element_type=jnp.float32)


def _lnf(y, bdsum, g, b):
    mean = _bdot(y, bdsum) * _INV_H
    c = y - mean
    v = _bdot(c * c, bdsum) * _INV_H
    return g * c * jax.lax.rsqrt(v + 1e-5) + b


def _layer(h, w, bdsum, aux_ref, i):
    b = aux_ref[3 * i:3 * i + 1, :]
    g = aux_ref[3 * i + 1:3 * i + 2, :]
    bb = aux_ref[3 * i + 2:3 * i + 3, :]
    return jax.nn.relu(_lnf(_bdot(h, w) + b, bdsum, g, bb))


def _in_embed_body(xf_ref, w_ref, aux_ref, out_ref):
    w = w_ref[0:LANES, :]
    bdsum = w_ref[LANES:2 * LANES, :]
    h = _bdot(xf_ref[...], w)
    h = h + aux_ref[0:1, :]
    out_ref[...] = jax.nn.relu(_lnf(h, bdsum, aux_ref[1:2, :], aux_ref[2:3, :]))


def _edge_hidden(gs, ge, w_ref, aux_ref):
    bdsum = w_ref[640:768, :]
    h = _bdot(gs, w_ref[0:128, :]) + _bdot(ge, w_ref[128:256, :])
    b = aux_ref[0:1, :]
    g = aux_ref[1:2, :]
    bb = aux_ref[2:3, :]
    h = jax.nn.relu(_lnf(h + b, bdsum, g, bb))
    h = _layer(h, w_ref[256:384, :], bdsum, aux_ref, 1)
    h = _layer(h, w_ref[384:512, :], bdsum, aux_ref, 2)
    return h


def _edge_logit_b(h, w_ref, aux_ref):
    return _bdot(h, w_ref[512:640, :]) + aux_ref[9:10, :]


def _make_edge_body(off):
    def body(gs_ref, ge_ref, w_ref, aux_ref, ws_ref, we_ref):
        i = pl.program_id(0)
        gs = gs_ref[...]
        ge = ge_ref[...]
        h = _edge_hidden(gs, ge, w_ref, aux_ref)
        e = jax.nn.sigmoid(_edge_logit_b(h, w_ref, aux_ref))
        r = lax.broadcasted_iota(jnp.int32, (BR, LANES), 0)
        l = lax.broadcasted_iota(jnp.int32, (BR, LANES), 1)
        eid = off + (i * BR + r) * FE + (l >> 4)
        mask = eid < N_EDGES
        ws_ref[...] = jnp.where(mask, e * gs, 0.0)
        we_ref[...] = jnp.where(mask, e * ge, 0.0)
    return body


def _make_final_edge_body(off):
    def body(gs_ref, ge_ref, we_ref, auxe_ref, wp_ref, auxp_ref,
             sel_ref, a_ref, ep_ref):
        i = pl.program_id(0)
        gs = gs_ref[...]
        ge = ge_ref[...]
        sel = sel_ref[...]
        ha = _edge_hidden(gs, ge, we_ref, auxe_ref)
        alog8 = jnp.dot(_edge_logit_b(ha, we_ref, auxe_ref), sel,
                        preferred_element_type=jnp.float32)
        a_ref[...] = jax.nn.sigmoid(alog8)
        hp = _edge_hidden(gs, ge, wp_ref, auxp_ref)
        ep8 = jnp.dot(_edge_logit_b(hp, wp_ref, auxp_ref), sel,
                      preferred_element_type=jnp.float32)
        r = lax.broadcasted_iota(jnp.int32, (BR, FE), 0)
        g = lax.broadcasted_iota(jnp.int32, (BR, FE), 1)
        eid = off + (i * BR + r) * FE + g
        ep_ref[...] = jnp.where(eid < N_EDGES, ep8, -jnp.inf)
    return body


def _node_mlp_body(mi0_ref, mi1_ref, mo0_ref, mo1_ref, h_ref, w_ref,
                   aux_ref, out_ref):
    mi = mi0_ref[0] + mi1_ref[0]
    mo = mo0_ref[0] + mo1_ref[0]
    h0 = h_ref[...]
    bdsum = w_ref[768:896, :]
    h = (_bdot(mi, w_ref[0:128, :]) + _bdot(mo, w_ref[128:256, :])
         + _bdot(h0, w_ref[256:384, :]))
    h = jax.nn.relu(_lnf(h + aux_ref[0:1, :], bdsum,
                         aux_ref[1:2, :], aux_ref[2:3, :]))
    h = _layer(h, w_ref[384:512, :], bdsum, aux_ref, 1)
    h = _layer(h, w_ref[512:640, :], bdsum, aux_ref, 2)
    h = _layer(h, w_ref[640:768, :], bdsum, aux_ref, 3)
    out_ref[...] = h + h0


def _gsum_body(h_ref, out_ref):
    i = pl.program_id(0)

    @pl.when(i == 0)
    def _():
        out_ref[...] = jnp.zeros_like(out_ref)

    r = lax.broadcasted_iota(jnp.int32, (BNR, LANES), 0)
    blk = jnp.where(i * BNR + r < NROWS, h_ref[...], 0.0)
    out_ref[...] += jnp.sum(blk, axis=0, keepdims=True)


def _glob_body(gsf_ref, t_ref, wcat_ref, aux_ref, w3_ref, b3_ref,
               graw_ref, gsig_ref):
    h = jnp.dot(gsf_ref[...], t_ref[...], preferred_element_type=jnp.float32)
    for i, (r0, r1) in enumerate(((0, 16), (16, 32), (32, 48))):
        w = wcat_ref[r0:r1, :]
        b = aux_ref[3 * i:3 * i + 1, :]
        g = aux_ref[3 * i + 1:3 * i + 2, :]
        bb = aux_ref[3 * i + 2:3 * i + 3, :]
        hlin = jnp.dot(h, w, preferred_element_type=jnp.float32) + b
        mu = jnp.mean(hlin, axis=-1, keepdims=True)
        va = jnp.mean((hlin - mu) * (hlin - mu), axis=-1, keepdims=True)
        h = jax.nn.relu(g * (hlin - mu) * jax.lax.rsqrt(va + 1e-5) + bb)
    graw = jnp.dot(h, w3_ref[...], preferred_element_type=jnp.float32) + b3_ref[...]
    graw_ref[...] = graw
    gsig_ref[...] = jax.nn.sigmoid(graw)


def _lsm_pass1_body(ep_ref, g00_ref, m_ref, s_ref, acc_ref):
    i = pl.program_id(0)

    @pl.when(i == 0)
    def _():
        acc_ref[0] = g00_ref[0, 0]
        acc_ref[1] = 1.0

    m_old = acc_ref[0]
    s_old = acc_ref[1]
    blk = ep_ref[...]
    bm = jnp.max(blk)
    m_new = jnp.maximum(m_old, bm)
    s_new = s_old * jnp.exp(m_old - m_new) + jnp.sum(jnp.exp(blk - m_new))
    acc_ref[0] = m_new
    acc_ref[1] = s_new

    @pl.when(i == pl.num_programs(0) - 1)
    def _():
        m_ref[0, 0] = m_new
        s_ref[0, 0] = s_new


def _lsm_pass2_body(ep_ref, g00_ref, m_ref, s_ref, lsm_ref, lsm0_ref):
    z = m_ref[0, 0] + jnp.log(s_ref[0, 0])
    lsm_ref[...] = ep_ref[...] - z

    @pl.when(pl.program_id(0) == 0)
    def _():
        lsm0_ref[0, 0] = g00_ref[0, 0] - z


def _bd(w):
    return jnp.kron(jnp.eye(FE, dtype=jnp.float32), w)


def _bdsum():
    return jnp.kron(jnp.eye(FE, dtype=jnp.float32),
                    jnp.ones((HID, HID), jnp.float32))


def _tile_rows(p, names):
    return jnp.stack([jnp.tile(v, FE) for v in names], axis=0)


def _pack_edge(p):
    w0 = p["l0"]["W"]
    head = jnp.outer(p["l3"]["W"][:, 0], jnp.ones((HID,), jnp.float32))
    wbig = jnp.concatenate([
        _bd(w0[:HID]), _bd(w0[HID:]), _bd(p["l1"]["W"]), _bd(p["l2"]["W"]),
        _bd(head), _bdsum()], axis=0)
    rows = []
    for i in range(3):
        rows += [p["l%d" % i]["b"], p["ln%d" % i]["g"], p["ln%d" % i]["b"]]
    rows.append(jnp.full((HID,), p["l3"]["b"][0], jnp.float32))
    aux = _tile_rows(p, rows)
    return wbig, aux


def _pack_node(p):
    w0 = p["l0"]["W"]
    wbig = jnp.concatenate([
        _bd(w0[0:HID]), _bd(w0[HID:2 * HID]), _bd(w0[2 * HID:3 * HID]),
        _bd(p["l1"]["W"]), _bd(p["l2"]["W"]), _bd(p["l3"]["W"]),
        _bdsum()], axis=0)
    rows = []
    for i in range(4):
        rows += [p["l%d" % i]["b"], p["ln%d" % i]["g"], p["ln%d" % i]["b"]]
    return wbig, _tile_rows(p, rows)


def _pack_glob(p):
    wcat = jnp.concatenate([p["l0"]["W"], p["l1"]["W"], p["l2"]["W"]], axis=0)
    rows = []
    for i in range(3):
        rows += [p["l%d" % i]["b"], p["ln%d" % i]["g"], p["ln%d" % i]["b"]]
    return wcat, jnp.stack(rows, axis=0), p["l3"]["W"], p["l3"]["b"][None, :]


_TC_PARAMS = pltpu.CompilerParams(dimension_semantics=("arbitrary",))


def _wspec(shape):
    return pl.BlockSpec(shape, lambda i: (0,) * len(shape))


def kernel(params, x, edge_index):
    start = edge_index[0]
    end = edge_index[1]
    zpad = jnp.zeros((EPAD - N_EDGES,), jnp.int32)
    startp = jnp.concatenate([start, zpad])
    endp = jnp.concatenate([end, zpad])

    idx_ga = jnp.concatenate([startp[:EA], endp[:EA]]).reshape(-1, 128)
    idx_gb = jnp.concatenate([startp[EA:], endp[EA:]]).reshape(-1, 128)
    ia_mi = endp[:EA].reshape(-1, 128)
    ia_mo = (startp[:EA] + MO_OFF).reshape(-1, 128)
    ib_mi = endp[EA:].reshape(-1, 128)
    ib_mo = (startp[EA:] + MO_OFF).reshape(-1, 128)
    zrows = jnp.zeros((ZROWS, HID), jnp.float32)

    wbig_e, aux_e = _pack_edge(params["edge"])
    wbig_p, aux_p = _pack_edge(params["prune"])
    wbig_n, aux_n = _pack_node(params["node"])
    wcat_g, aux_g, w3_g, b3_g = _pack_glob(params["glob"])
    sel = jnp.kron(jnp.eye(FE, dtype=jnp.float32),
                   jnp.zeros((HID, 1), jnp.float32).at[0, 0].set(1.0))
    tmat = jnp.tile(jnp.eye(HID, dtype=jnp.float32), (FE, 1))

    xf = jnp.pad(x, ((0, 0), (0, HID - 3))).reshape(NROWS, LANES)
    w16 = jnp.pad(params["in_lin"]["W"], ((0, HID - 3), (0, 0)))
    wbig_i = jnp.concatenate([_bd(w16), _bdsum()], axis=0)
    aux_i = _tile_rows(None, [params["in_lin"]["b"], params["in_ln"]["g"],
                              params["in_ln"]["b"]])

    hf = pl.pallas_call(
        _in_embed_body,
        grid=(NBN,),
        in_specs=[pl.BlockSpec((BNR, LANES), lambda i: (i, 0)),
                  _wspec((2 * LANES, LANES)), _wspec((3, LANES))],
        out_specs=pl.BlockSpec((BNR, LANES), lambda i: (i, 0)),
        out_shape=jax.ShapeDtypeStruct((NROWS, LANES), jnp.float32),
        compiler_params=_TC_PARAMS,
    )(xf, wbig_i, aux_i)

    def edge_call(nblk, off):
        return pl.pallas_call(
            _make_edge_body(off),
            grid=(nblk,),
            in_specs=[pl.BlockSpec((BR, LANES), lambda i: (i, 0)),
                      pl.BlockSpec((BR, LANES), lambda i: (nblk + i, 0)),
                      _wspec((768, LANES)), _wspec((10, LANES))],
            out_specs=[pl.BlockSpec((BR, LANES), lambda i: (i, 0)),
                       pl.BlockSpec((BR, LANES), lambda i: (i, 0))],
            out_shape=[jax.ShapeDtypeStruct((nblk * BR, LANES), jnp.float32),
                       jax.ShapeDtypeStruct((nblk * BR, LANES), jnp.float32)],
            compiler_params=_TC_PARAMS,
        )

    edge_a = edge_call(EAF // BR, 0)
    edge_b = edge_call(EBF // BR, EA)

    node_mlp = pl.pallas_call(
        _node_mlp_body,
        grid=(NBN,),
        in_specs=[pl.BlockSpec((1, BNR, LANES), lambda i: (0, i, 0)),
                  pl.BlockSpec((1, BNR, LANES), lambda i: (1, i, 0)),
                  pl.BlockSpec((1, BNR, LANES), lambda i: (0, MO_BLK + i, 0)),
                  pl.BlockSpec((1, BNR, LANES), lambda i: (1, MO_BLK + i, 0)),
                  pl.BlockSpec((BNR, LANES), lambda i: (i, 0)),
                  _wspec((896, LANES)), _wspec((12, LANES))],
        out_specs=pl.BlockSpec((BNR, LANES), lambda i: (i, 0)),
        out_shape=jax.ShapeDtypeStruct((NROWS, LANES), jnp.float32),
        compiler_params=_TC_PARAMS,
    )

    def final_call(nblk, off):
        return pl.pallas_call(
            _make_final_edge_body(off),
            grid=(nblk,),
            in_specs=[pl.BlockSpec((BR, LANES), lambda i: (i, 0)),
                      pl.BlockSpec((BR, LANES), lambda i: (nblk + i, 0)),
                      _wspec((768, LANES)), _wspec((10, LANES)),
                      _wspec((768, LANES)), _wspec((10, LANES)),
                      _wspec((LANES, FE))],
            out_specs=[pl.BlockSpec((BR, FE), lambda i: (i, 0)),
                       pl.BlockSpec((BR, FE), lambda i: (i, 0))],
            out_shape=[jax.ShapeDtypeStruct((nblk * BR, FE), jnp.float32),
                       jax.ShapeDtypeStruct((nblk * BR, FE), jnp.float32)],
            compiler_params=_TC_PARAMS,
        )

    for _ in range(3):
        h_rows = hf.reshape(N_NODES, HID)
        ga = _sc_gather(h_rows, idx_ga, 2 * EA).reshape(2 * EAF, LANES)
        gb = _sc_gather(h_rows, idx_gb, 2 * EB).reshape(2 * EBF, LANES)
        wsa, wea = edge_a(ga, ga, wbig_e, aux_e)
        wsb, web = edge_b(gb, gb, wbig_e, aux_e)
        part = _sc_scatter(wsa.reshape(EA, HID), wea.reshape(EA, HID),
                           wsb.reshape(EB, HID), web.reshape(EB, HID),
                           ia_mi, ia_mo, ib_mi, ib_mo, zrows)
        pf = part.reshape(NC, PROWS, LANES)
        hf = node_mlp(pf, pf, pf, pf, hf, wbig_n, aux_n)

    h_rows = hf.reshape(N_NODES, HID)
    ga = _sc_gather(h_rows, idx_ga, 2 * EA).reshape(2 * EAF, LANES)
    gb = _sc_gather(h_rows, idx_gb, 2 * EB).reshape(2 * EBF, LANES)
    a_sig_a, ep_a = final_call(EAF // BR, 0)(
        ga, ga, wbig_e, aux_e, wbig_p, aux_p, sel)
    a_sig_b, ep_b = final_call(EBF // BR, EA)(
        gb, gb, wbig_e, aux_e, wbig_p, aux_p, sel)
    a_sig = jnp.concatenate([a_sig_a, a_sig_b], axis=0)
    ep = jnp.concatenate([ep_a, ep_b], axis=0)

    gsf = pl.pallas_call(
        _gsum_body,
        grid=(NBN,),
        in_specs=[pl.BlockSpec((BNR, LANES), lambda i: (i, 0))],
        out_specs=pl.BlockSpec((1, LANES), lambda i: (0, 0)),
        out_shape=jax.ShapeDtypeStruct((1, LANES), jnp.float32),
        compiler_params=_TC_PARAMS,
    )(hf)

    graw, gsig = pl.pallas_call(
        _glob_body,
        out_shape=[jax.ShapeDtypeStruct((1, 3), jnp.float32),
                   jax.ShapeDtypeStruct((1, 3), jnp.float32)],
    )(gsf, tmat, wcat_g, aux_g, w3_g, b3_g)

    g00 = graw[:, 0:1]
    _sspec = pl.BlockSpec(memory_space=pltpu.SMEM)
    BSR = 4096
    m, s = pl.pallas_call(
        _lsm_pass1_body,
        grid=(EROWS // BSR,),
        in_specs=[pl.BlockSpec((BSR, FE), lambda i: (i, 0)), _sspec],
        out_specs=[_sspec, _sspec],
        out_shape=[jax.ShapeDtypeStruct((1, 1), jnp.float32),
                   jax.ShapeDtypeStruct((1, 1), jnp.float32)],
        scratch_shapes=[pltpu.SMEM((2,), jnp.float32)],
        compiler_params=_TC_PARAMS,
    )(ep, g00)

    lsm, lsm0 = pl.pallas_call(
        _lsm_pass2_body,
        grid=(EROWS // BSR,),
        in_specs=[pl.BlockSpec((BSR, FE), lambda i: (i, 0)), _sspec,
                  _sspec, _sspec],
        out_specs=[pl.BlockSpec((BSR, FE), lambda i: (i, 0)), _sspec],
        out_shape=[jax.ShapeDtypeStruct((EROWS, FE), jnp.float32),
                   jax.ShapeDtypeStruct((1, 1), jnp.float32)],
        compiler_params=_TC_PARAMS,
    )(ep, g00, m, s)

    out_lsm = jnp.concatenate([lsm0[0], lsm.reshape(-1)[:N_EDGES]])
    return (out_lsm, gsig[0, 1:2], a_sig.reshape(-1)[:N_EDGES], gsig[0, 2:3])
